# baseline (speedup 1.0000x reference)
import functools

import jax
import jax.numpy as jnp
from jax import lax
from jax.experimental import pallas as pl
from jax.experimental.pallas import tpu as pltpu
from jax.experimental.pallas import tpu_sc as plsc


def _cdiv(a, b):
    return (a + b - 1) // b


def _node_pad(n):
    return _cdiv(n + 1, 128) * 128


def _make_pi_kernel(n_nodes, e1, e1p, pi_iters):
    npad = _cdiv(n_nodes, 1024) * 1024
    per_tile = e1p // 16
    c_chunks = per_tile // 16
    slice_sz = npad // 16
    sum_chunks = n_nodes // 16
    mesh = plsc.VectorSubcoreMesh(core_axis_name="c", subcore_axis_name="s")

    @functools.partial(
        pl.kernel,
        out_type=[jax.ShapeDtypeStruct((npad,), jnp.float32),
                  jax.ShapeDtypeStruct((npad,), jnp.float32)],
        mesh=mesh,
        compiler_params=pltpu.CompilerParams(needs_layout_passes=False),
        scratch_types=[
            pltpu.VMEM((per_tile,), jnp.int32),
            pltpu.VMEM((per_tile,), jnp.int32),
            pltpu.VMEM((per_tile,), jnp.float32),
            pltpu.VMEM((npad,), jnp.float32),
            pltpu.VMEM((npad,), jnp.float32),
            pltpu.VMEM((npad,), jnp.float32),
            pltpu.VMEM((npad,), jnp.float32),
            pltpu.VMEM((16, slice_sz), jnp.float32),
            pltpu.VMEM((slice_sz,), jnp.float32),
            pltpu.VMEM((16,), jnp.float32),
            pltpu.VMEM_SHARED((16, npad), jnp.float32),
            pltpu.VMEM_SHARED((npad,), jnp.float32),
        ],
    )
    def pi_kernel(row_h, col_h, alpha_h, a_h, b_h,
                  row_v, col_v, p_v, pi_v, lacc_v, r_v, deginv_v, tmp_v,
                  red_v, alpha_v, acc_all, pi_sh):
        c = lax.axis_index("c")
        w = lax.axis_index("s")
        iota16 = lax.broadcasted_iota(jnp.int32, (16,), 0)
        zero16 = jnp.zeros((16,), jnp.float32)

        pltpu.sync_copy(row_h.at[w], row_v)
        pltpu.sync_copy(col_h.at[w], col_v)
        pltpu.sync_copy(alpha_h, alpha_v)
        va = alpha_v[...]

        def zero_lacc(_i, carry):
            lacc_v[pl.ds(_i * 16, 16)] = zero16
            return carry

        def allreduce_to_pi():
            pltpu.sync_copy(lacc_v, acc_all.at[w])
            plsc.subcore_barrier()
            pltpu.sync_copy(acc_all.at[:, pl.ds(w * slice_sz, slice_sz)],
                            tmp_v)

            def addc(i, cc):
                r = tmp_v[0, pl.ds(i * 16, 16)]
                for t in range(1, 16):
                    r = r + tmp_v[t, pl.ds(i * 16, 16)]
                red_v[pl.ds(i * 16, 16)] = r
                return cc
            lax.fori_loop(0, slice_sz // 16, addc, 0)
            pltpu.sync_copy(red_v, pi_sh.at[pl.ds(w * slice_sz, slice_sz)])
            plsc.subcore_barrier()
            pltpu.sync_copy(pi_sh, pi_v)

        lax.fori_loop(0, npad // 16, zero_lacc, 0)

        def deg_body(b, carry):
            gid = w * per_tile + b * 16 + iota16
            ind = jnp.where(gid < e1, 1.0, 0.0).astype(jnp.float32)
            idx = row_v[pl.ds(b * 16, 16)]
            plsc.addupdate_scatter(lacc_v, [idx], ind)
            return carry
        lax.fori_loop(0, c_chunks, deg_body, 0)
        allreduce_to_pi()

        def dib(i, carry):
            dg = pi_v[pl.ds(i * 16, 16)]
            deginv_v[pl.ds(i * 16, 16)] = jnp.where(
                dg > 0.0, 1.0 / jnp.maximum(dg, 1.0e-30), 0.0)
            return carry
        lax.fori_loop(0, npad // 16, dib, 0)

        def p_body(b, carry):
            gid = w * per_tile + b * 16 + iota16
            ind = jnp.where(gid < e1, 1.0, 0.0).astype(jnp.float32)
            idx = row_v[pl.ds(b * 16, 16)]
            dg = plsc.load_gather(deginv_v, [idx])
            p_v[pl.ds(b * 16, 16)] = ind * dg
            return carry
        lax.fori_loop(0, c_chunks, p_body, 0)

        inv_n = jnp.float32(1.0 / n_nodes)

        def init_body(i, carry):
            pi_v[pl.ds(i * 16, 16)] = jnp.full((16,), inv_n, jnp.float32)
            return carry
        lax.fori_loop(0, npad // 16, init_body, 0)

        def iter_body(_t, carry):
            lax.fori_loop(0, npad // 16, zero_lacc, 0)

            def vb(b, cc):
                idx = row_v[pl.ds(b * 16, 16)]
                g = plsc.load_gather(pi_v, [idx])
                v = g * p_v[pl.ds(b * 16, 16)]
                cidx = col_v[pl.ds(b * 16, 16)]
                plsc.addupdate_scatter(lacc_v, [cidx], v)
                return cc
            lax.fori_loop(0, c_chunks, vb, 0)
            allreduce_to_pi()

            def ab(i, acc16):
                v = pi_v[pl.ds(i * 16, 16)]
                v2 = (1.0 - va) * v + va * inv_n
                pi_v[pl.ds(i * 16, 16)] = v2
                return acc16 + v2
            s16 = lax.fori_loop(0, sum_chunks, ab, zero16)
            total = jnp.sum(s16)

            def nb(i, cc):
                pi_v[pl.ds(i * 16, 16)] = pi_v[pl.ds(i * 16, 16)] / total
                return cc
            lax.fori_loop(0, sum_chunks, nb, 0)
            return carry
        lax.fori_loop(0, pi_iters, iter_body, 0)

        def rb(i, carry):
            v = pi_v[pl.ds(i * 16, 16)]
            x = jnp.maximum(v, 1e-12)
            ii = plsc.bitcast(x, jnp.int32)
            ii = jnp.int32(0x5F3759DF) - (ii >> 1)
            y = plsc.bitcast(ii, jnp.float32)
            y = y * (1.5 - 0.5 * x * y * y)
            y = y * (1.5 - 0.5 * x * y * y)
            y = y * (1.5 - 0.5 * x * y * y)
            r_v[pl.ds(i * 16, 16)] = y
            pi_v[pl.ds(i * 16, 16)] = x * y
            return carry
        lax.fori_loop(0, npad // 16, rb, 0)

        def awb(i, carry):
            o = w * slice_sz + i * 16
            red_v[pl.ds(i * 16, 16)] = (
                0.5 * pi_v[pl.ds(o, 16)] * deginv_v[pl.ds(o, 16)])
            return carry
        lax.fori_loop(0, slice_sz // 16, awb, 0)

        @pl.when(c == 0)
        def _():
            pltpu.sync_copy(red_v, a_h.at[pl.ds(w * slice_sz, slice_sz)])
            pltpu.sync_copy(r_v.at[pl.ds(w * slice_sz, slice_sz)],
                            b_h.at[pl.ds(w * slice_sz, slice_sz)])

    return pi_kernel


_CH = 128


def _make_prop_kernel(n_nodes, d, e1p):
    per_tile = e1p // 16
    t_chunks = per_tile // _CH
    npad = _node_pad(n_nodes)
    rows_per_tile = npad // 16
    mesh = plsc.VectorSubcoreMesh(core_axis_name="c", subcore_axis_name="s")

    @functools.partial(
        pl.kernel,
        out_type=jax.ShapeDtypeStruct((2 * npad, d), jnp.float32),
        mesh=mesh,
        compiler_params=pltpu.CompilerParams(needs_layout_passes=False),
        scratch_types=[
            pltpu.VMEM((2, 128), jnp.int32),
            pltpu.VMEM((2, 128), jnp.int32),
            pltpu.VMEM((2, 128), jnp.int32),
            pltpu.VMEM((2, 128), jnp.int32),
            pltpu.VMEM((128, d), jnp.float32),
            pltpu.VMEM((128, d), jnp.float32),
            pltpu.VMEM((128, d), jnp.float32),
            pltpu.VMEM_SHARED((npad, d), jnp.float32),
            pltpu.SemaphoreType.DMA,
            pltpu.SemaphoreType.DMA,
            pltpu.SemaphoreType.DMA,
            pltpu.SemaphoreType.DMA,
            pltpu.SemaphoreType.DMA,
            pltpu.SemaphoreType.DMA,
            pltpu.SemaphoreType.DMA,
            pltpu.SemaphoreType.DMA,
            pltpu.SemaphoreType.DMA,
            pltpu.SemaphoreType.DMA,
        ],
    )
    def prop_kernel(ed_h, hab_h, out_h,
                    e0, e1, e2, e3, r0, r1, r2, acc,
                    es0, es1, es2, es3, gs0, gs1, gs2, ss0, ss1, ss2):
        c = lax.axis_index("c")
        s = lax.axis_index("s")
        g = c * 16 + s
        zero16 = jnp.zeros((16,), jnp.float32)
        ebufs = [e0, e1, e2, e3]
        esems = [es0, es1, es2, es3]
        rbufs = [r0, r1, r2]
        gsems = [gs0, gs1, gs2]
        ssems = [ss0, ss1, ss2]
        T = t_chunks

        def zb(r, carry):
            for i in range(d // 16):
                r0[r, pl.ds(i * 16, 16)] = zero16
            return carry
        lax.fori_loop(0, 128, zb, 0)
        row0 = s * rows_per_tile
        off = 0
        while off < rows_per_tile:
            sz = min(128, rows_per_tile - off)
            pltpu.sync_copy(r0.at[pl.ds(0, sz), :],
                            acc.at[pl.ds(row0 + off, sz), :])
            off += sz
        plsc.subcore_barrier()

        def drain_scatter(b):
            pltpu.make_async_copy(
                rbufs[b], acc.at[pl.ds(0, 128)], ssems[b]).wait()

        def drain_gather(b):
            pltpu.make_async_copy(
                hab_h.at[pl.ds(0, 128)], rbufs[b], gsems[b]).wait()

        pltpu.async_copy(ed_h.at[g].at[0], e0, es0).wait()
        pltpu.async_copy(ed_h.at[g].at[1], e1, es1)
        pltpu.async_copy(hab_h.at[e0.at[0]], r0, gs0)

        def step(st, carry):
            j0 = st * 12
            for u in range(12):
                j = j0 + u
                rb = u % 3
                nrb = (u + 1) % 3
                eu = u % 4
                nu = (u + 1) % 4
                pu = (u + 2) % 4
                drain_gather(rb)
                @pl.when(jnp.logical_and(j >= 2, j + 1 < T))
                def _():
                    drain_scatter(nrb)
                @pl.when(j + 2 < T)
                def _():
                    pltpu.async_copy(ed_h.at[g].at[j + 2], ebufs[pu],
                                     esems[pu])
                @pl.when(j + 1 < T)
                def _():
                    pltpu.make_async_copy(ed_h.at[g].at[0], ebufs[nu],
                                          esems[nu]).wait()
                    pltpu.async_copy(hab_h.at[ebufs[nu].at[0]], rbufs[nrb],
                                     gsems[nrb])
                pltpu.async_copy(rbufs[rb], acc.at[ebufs[eu].at[1]],
                                 ssems[rb], add=True)
            return carry
        lax.fori_loop(0, T // 12, step, 0)
        drain_scatter((T - 3) % 3)
        drain_scatter((T - 2) % 3)
        drain_scatter((T - 1) % 3)
        plsc.subcore_barrier()

        pltpu.sync_copy(
            acc.at[pl.ds(row0, rows_per_tile), :],
            out_h.at[pl.ds(c * npad + row0, rows_per_tile), :])

    return prop_kernel


_BLK = 1000


def _mm2_body(x_ref, w2_ref, wl_ref, bl_ref, a_ref, b_ref,
              h2_ref, o0_ref):
    xb = x_ref[...]
    av = a_ref[...]
    bv = b_ref[...]
    h2 = lax.dot_general(xb, w2_ref[...], (((1,), (0,)), ((), ())),
                         preferred_element_type=jnp.float32)
    h2_ref[0] = h2 * av
    h2_ref[1] = h2 * bv
    o0_ref[...] = lax.dot_general(
        xb, wl_ref[...], (((1,), (1,)), ((), ())),
        preferred_element_type=jnp.float32) + bl_ref[...]


def _mm2(x, w2, wl, bl, a, b, npad):
    n, d = x.shape
    grid = n // _BLK
    return pl.pallas_call(
        _mm2_body,
        grid=(grid,),
        in_specs=[
            pl.BlockSpec((_BLK, d), lambda i: (i, 0)),
            pl.BlockSpec((d, d), lambda i: (0, 0)),
            pl.BlockSpec((d, d), lambda i: (0, 0)),
            pl.BlockSpec((1, d), lambda i: (0, 0)),
            pl.BlockSpec((_BLK, 1), lambda i: (i, 0)),
            pl.BlockSpec((_BLK, 1), lambda i: (i, 0)),
        ],
        out_specs=[
            pl.BlockSpec((2, _BLK, d), lambda i: (0, i, 0)),
            pl.BlockSpec((_BLK, d), lambda i: (i, 0)),
        ],
        out_shape=[
            jax.ShapeDtypeStruct((2, npad, d), jnp.float32),
            jax.ShapeDtypeStruct((n, d), jnp.float32),
        ],
    )(x, w2, wl, bl, a, b)


def _zmid_body(x_ref, w1_ref, u_ref, a_ref, b_ref, o_ref):
    av = a_ref[...]
    bv = b_ref[...]
    h1 = lax.dot_general(x_ref[...], w1_ref[...], (((1,), (0,)), ((), ())),
                         preferred_element_type=jnp.float32)
    z = h1 + bv * u_ref[0] + av * u_ref[1]
    o_ref[0] = z * av
    o_ref[1] = z * bv


def _zmid(x, w1, upair, a, b):
    n, d = x.shape
    npad = upair.shape[1]
    grid = n // _BLK
    return pl.pallas_call(
        _zmid_body,
        grid=(grid,),
        in_specs=[
            pl.BlockSpec((_BLK, d), lambda i: (i, 0)),
            pl.BlockSpec((d, d), lambda i: (0, 0)),
            pl.BlockSpec((2, _BLK, d), lambda i: (0, i, 0)),
            pl.BlockSpec((_BLK, 1), lambda i: (i, 0)),
            pl.BlockSpec((_BLK, 1), lambda i: (i, 0)),
        ],
        out_specs=pl.BlockSpec((2, _BLK, d), lambda i: (0, i, 0)),
        out_shape=jax.ShapeDtypeStruct((2, npad, d), jnp.float32),
    )(x, w1, upair, a, b)


def _fin_body(o0_ref, y_ref, a_ref, b_ref, c1_ref, c2_ref, o_ref):
    o_ref[...] = (o0_ref[...]
                  + b_ref[...] * y_ref[0] + a_ref[...] * y_ref[1]
                  + c1_ref[...] + c2_ref[...])


def _fin(o0, y, a, b, c1, c2):
    n, d = o0.shape
    grid = n // _BLK
    return pl.pallas_call(
        _fin_body,
        grid=(grid,),
        in_specs=[
            pl.BlockSpec((_BLK, d), lambda i: (i, 0)),
            pl.BlockSpec((2, _BLK, d), lambda i: (0, i, 0)),
            pl.BlockSpec((_BLK, 1), lambda i: (i, 0)),
            pl.BlockSpec((_BLK, 1), lambda i: (i, 0)),
            pl.BlockSpec((1, d), lambda i: (0, 0)),
            pl.BlockSpec((1, d), lambda i: (0, 0)),
        ],
        out_specs=pl.BlockSpec((_BLK, d), lambda i: (i, 0)),
        out_shape=jax.ShapeDtypeStruct((n, d), jnp.float32),
    )(o0, y, a, b, c1, c2)


def kernel(x, alpha, lin_w, lin_b, conv1_w, conv1_b, conv2_w, conv2_b,
           edge_index):
    n, d = x.shape
    e = edge_index.shape[1]
    e1 = e + n
    e1p = _cdiv(e1, 16 * 12 * _CH) * (16 * 12 * _CH)
    pi_iters = 20
    npad = _node_pad(n)
    t_chunks = e1p // 16 // _CH

    loops = jnp.arange(n, dtype=jnp.int32)
    row = jnp.concatenate([edge_index[0], loops])
    col = jnp.concatenate([edge_index[1], loops])
    row_p = jnp.pad(row, (0, e1p - e1)).reshape(16, t_chunks, _CH)
    col_p = jnp.pad(col, (0, e1p - e1)).reshape(16, t_chunks, _CH)
    alpha16 = jnp.full((16,), alpha, jnp.float32)

    pi_kernel = _make_pi_kernel(n, e1, e1p, pi_iters)
    a_full, b_full = pi_kernel(row_p.reshape(16, -1), col_p.reshape(16, -1),
                               alpha16)
    a2 = a_full[:n].reshape(n, 1)
    b2 = b_full[:n].reshape(n, 1)

    shp = (16, t_chunks, _CH)
    src1 = row_p
    dst1 = jnp.pad(col, (0, e1p - e1),
                   constant_values=npad - 1).reshape(shp)
    src2 = col_p + npad
    dst2 = jnp.pad(row, (0, e1p - e1),
                   constant_values=npad - 1).reshape(shp)
    ed = jnp.concatenate([
        jnp.stack([src1, dst1], axis=2),
        jnp.stack([src2, dst2], axis=2),
    ], axis=0)

    prop = _make_prop_kernel(n, d, e1p)
    c1 = conv1_b.reshape(1, d)
    c2 = conv2_b.reshape(1, d)
    bl = lin_b.reshape(1, d)

    xc = x
    for _ in range(2):
        h2ab, o0 = _mm2(xc, conv2_w, lin_w, bl, a2, b2, npad)
        u = prop(ed, h2ab.reshape(2 * npad, d)).reshape(2, npad, d)
        zab = _zmid(xc, conv1_w, u, a2, b2)
        y = prop(ed, zab.reshape(2 * npad, d)).reshape(2, npad, d)
        xc = _fin(o0, y, a2, b2, c1, c2)
    return xc


# speedup vs baseline: 1.7186x; 1.7186x over previous
import functools

import jax
import jax.numpy as jnp
from jax import lax
from jax.experimental import pallas as pl
from jax.experimental.pallas import tpu as pltpu
from jax.experimental.pallas import tpu_sc as plsc


def _cdiv(a, b):
    return (a + b - 1) // b


def _make_pi_kernel(n_nodes, e1, e1p, pi_iters):
    npad = _cdiv(n_nodes, 1024) * 1024
    per_tile = e1p // 16
    c_chunks = per_tile // 16
    slice_sz = npad // 16
    sum_chunks = n_nodes // 16
    mesh = plsc.VectorSubcoreMesh(core_axis_name="c", subcore_axis_name="s")

    @functools.partial(
        pl.kernel,
        out_type=[jax.ShapeDtypeStruct((npad,), jnp.float32),
                  jax.ShapeDtypeStruct((npad,), jnp.float32)],
        mesh=mesh,
        compiler_params=pltpu.CompilerParams(needs_layout_passes=False),
        scratch_types=[
            pltpu.VMEM((per_tile,), jnp.int32),
            pltpu.VMEM((per_tile,), jnp.int32),
            pltpu.VMEM((per_tile,), jnp.float32),
            pltpu.VMEM((npad,), jnp.float32),
            pltpu.VMEM((npad,), jnp.float32),
            pltpu.VMEM((npad,), jnp.float32),
            pltpu.VMEM((npad,), jnp.float32),
            pltpu.VMEM((16, slice_sz), jnp.float32),
            pltpu.VMEM((slice_sz,), jnp.float32),
            pltpu.VMEM((16,), jnp.float32),
            pltpu.VMEM_SHARED((16, npad), jnp.float32),
            pltpu.VMEM_SHARED((npad,), jnp.float32),
        ],
    )
    def pi_kernel(row_h, col_h, alpha_h, a_h, b_h,
                  row_v, col_v, p_v, pi_v, lacc_v, r_v, deginv_v, tmp_v,
                  red_v, alpha_v, acc_all, pi_sh):
        c = lax.axis_index("c")
        w = lax.axis_index("s")
        iota16 = lax.broadcasted_iota(jnp.int32, (16,), 0)
        zero16 = jnp.zeros((16,), jnp.float32)

        pltpu.sync_copy(row_h.at[w], row_v)
        pltpu.sync_copy(col_h.at[w], col_v)
        pltpu.sync_copy(alpha_h, alpha_v)
        va = alpha_v[...]

        def zero_lacc(_i, carry):
            lacc_v[pl.ds(_i * 16, 16)] = zero16
            return carry

        def allreduce_to_pi():
            pltpu.sync_copy(lacc_v, acc_all.at[w])
            plsc.subcore_barrier()
            pltpu.sync_copy(acc_all.at[:, pl.ds(w * slice_sz, slice_sz)],
                            tmp_v)

            def addc(i, cc):
                r = tmp_v[0, pl.ds(i * 16, 16)]
                for t in range(1, 16):
                    r = r + tmp_v[t, pl.ds(i * 16, 16)]
                red_v[pl.ds(i * 16, 16)] = r
                return cc
            lax.fori_loop(0, slice_sz // 16, addc, 0)
            pltpu.sync_copy(red_v, pi_sh.at[pl.ds(w * slice_sz, slice_sz)])
            plsc.subcore_barrier()
            pltpu.sync_copy(pi_sh, pi_v)

        lax.fori_loop(0, npad // 16, zero_lacc, 0)

        def deg_body(b, carry):
            gid = w * per_tile + b * 16 + iota16
            ind = jnp.where(gid < e1, 1.0, 0.0).astype(jnp.float32)
            idx = row_v[pl.ds(b * 16, 16)]
            plsc.addupdate_scatter(lacc_v, [idx], ind)
            return carry
        lax.fori_loop(0, c_chunks, deg_body, 0)
        allreduce_to_pi()

        def dib(i, carry):
            dg = pi_v[pl.ds(i * 16, 16)]
            deginv_v[pl.ds(i * 16, 16)] = jnp.where(
                dg > 0.0, 1.0 / jnp.maximum(dg, 1.0e-30), 0.0)
            return carry
        lax.fori_loop(0, npad // 16, dib, 0)

        def p_body(b, carry):
            gid = w * per_tile + b * 16 + iota16
            ind = jnp.where(gid < e1, 1.0, 0.0).astype(jnp.float32)
            idx = row_v[pl.ds(b * 16, 16)]
            dg = plsc.load_gather(deginv_v, [idx])
            p_v[pl.ds(b * 16, 16)] = ind * dg
            return carry
        lax.fori_loop(0, c_chunks, p_body, 0)

        inv_n = jnp.float32(1.0 / n_nodes)

        def init_body(i, carry):
            pi_v[pl.ds(i * 16, 16)] = jnp.full((16,), inv_n, jnp.float32)
            return carry
        lax.fori_loop(0, npad // 16, init_body, 0)

        def iter_body(_t, carry):
            lax.fori_loop(0, npad // 16, zero_lacc, 0)

            def vb(b, cc):
                idx = row_v[pl.ds(b * 16, 16)]
                g = plsc.load_gather(pi_v, [idx])
                v = g * p_v[pl.ds(b * 16, 16)]
                cidx = col_v[pl.ds(b * 16, 16)]
                plsc.addupdate_scatter(lacc_v, [cidx], v)
                return cc
            lax.fori_loop(0, c_chunks, vb, 0)
            allreduce_to_pi()

            def ab(i, acc16):
                v = pi_v[pl.ds(i * 16, 16)]
                v2 = (1.0 - va) * v + va * inv_n
                pi_v[pl.ds(i * 16, 16)] = v2
                return acc16 + v2
            s16 = lax.fori_loop(0, sum_chunks, ab, zero16)
            total = jnp.sum(s16)

            def nb(i, cc):
                pi_v[pl.ds(i * 16, 16)] = pi_v[pl.ds(i * 16, 16)] / total
                return cc
            lax.fori_loop(0, sum_chunks, nb, 0)
            return carry
        lax.fori_loop(0, pi_iters, iter_body, 0)

        def rb(i, carry):
            v = pi_v[pl.ds(i * 16, 16)]
            x = jnp.maximum(v, 1e-12)
            ii = plsc.bitcast(x, jnp.int32)
            ii = jnp.int32(0x5F3759DF) - (ii >> 1)
            y = plsc.bitcast(ii, jnp.float32)
            y = y * (1.5 - 0.5 * x * y * y)
            y = y * (1.5 - 0.5 * x * y * y)
            y = y * (1.5 - 0.5 * x * y * y)
            r_v[pl.ds(i * 16, 16)] = y
            pi_v[pl.ds(i * 16, 16)] = x * y
            return carry
        lax.fori_loop(0, npad // 16, rb, 0)

        def awb(i, carry):
            o = w * slice_sz + i * 16
            red_v[pl.ds(i * 16, 16)] = (
                0.5 * pi_v[pl.ds(o, 16)] * deginv_v[pl.ds(o, 16)])
            return carry
        lax.fori_loop(0, slice_sz // 16, awb, 0)

        @pl.when(c == 0)
        def _():
            pltpu.sync_copy(red_v, a_h.at[pl.ds(w * slice_sz, slice_sz)])
            pltpu.sync_copy(r_v.at[pl.ds(w * slice_sz, slice_sz)],
                            b_h.at[pl.ds(w * slice_sz, slice_sz)])

    return pi_kernel


_CH = 128


def _make_prop_kernel(n_nodes, d, e1p):
    per_tile = e1p // 16
    t_chunks = per_tile // _CH
    npad = _cdiv(n_nodes, 1024) * 1024
    rows_per_tile = npad // 16
    mesh = plsc.VectorSubcoreMesh(core_axis_name="c", subcore_axis_name="s")

    @functools.partial(
        pl.kernel,
        out_type=jax.ShapeDtypeStruct((2 * npad, d), jnp.float32),
        mesh=mesh,
        compiler_params=pltpu.CompilerParams(needs_layout_passes=False),
        scratch_types=[
            pltpu.VMEM((2, 128), jnp.int32),
            pltpu.VMEM((2, 128), jnp.int32),
            pltpu.VMEM((2, 128), jnp.int32),
            pltpu.VMEM((2, 128), jnp.int32),
            pltpu.VMEM((128, d), jnp.float32),
            pltpu.VMEM((128, d), jnp.float32),
            pltpu.VMEM_SHARED((npad, d), jnp.float32),
            pltpu.SemaphoreType.DMA,
            pltpu.SemaphoreType.DMA,
            pltpu.SemaphoreType.DMA,
            pltpu.SemaphoreType.DMA,
            pltpu.SemaphoreType.DMA,
            pltpu.SemaphoreType.DMA,
            pltpu.SemaphoreType.DMA,
            pltpu.SemaphoreType.DMA,
        ],
    )
    def prop_kernel(ed_h, hab_h, out_h,
                    e0, e1, e2, e3, r0, r1, acc,
                    es0, es1, es2, es3, gs0, gs1, ss0, ss1):
        c = lax.axis_index("c")
        s = lax.axis_index("s")
        g = c * 16 + s
        zero16 = jnp.zeros((16,), jnp.float32)
        ebufs = [e0, e1, e2, e3]
        esems = [es0, es1, es2, es3]
        rbufs = [r0, r1]
        gsems = [gs0, gs1]
        ssems = [ss0, ss1]
        T = t_chunks

        def zb(r, carry):
            for i in range(d // 16):
                r0[r, pl.ds(i * 16, 16)] = zero16
            return carry
        lax.fori_loop(0, 128, zb, 0)
        row0 = s * rows_per_tile
        for t in range(rows_per_tile // 128):
            pltpu.sync_copy(r0, acc.at[pl.ds(row0 + t * 128, 128), :])
        plsc.subcore_barrier()

        def drain_scatter(b):
            pltpu.make_async_copy(
                rbufs[b], acc.at[pl.ds(0, 128)], ssems[b]).wait()

        def drain_gather(b):
            pltpu.make_async_copy(
                hab_h.at[pl.ds(0, 128)], rbufs[b], gsems[b]).wait()

        pltpu.async_copy(ed_h.at[g].at[0], e0, es0).wait()
        pltpu.async_copy(ed_h.at[g].at[1], e1, es1)
        pltpu.async_copy(hab_h.at[e0.at[0]], r0, gs0)

        def step(st, carry):
            j0 = st * 4
            for u in range(4):
                j = j0 + u
                rb = u % 2
                nu = (u + 1) % 4
                pu = (u + 2) % 4
                @pl.when(j + 2 < T)
                def _():
                    pltpu.async_copy(ed_h.at[g].at[j + 2], ebufs[pu],
                                     esems[pu])
                drain_gather(rb)
                @pl.when(j + 1 < T)
                def _():
                    pltpu.make_async_copy(ed_h.at[g].at[0], ebufs[nu],
                                          esems[nu]).wait()

                    @pl.when(j >= 1)
                    def _():
                        drain_scatter(1 - rb)
                    pltpu.async_copy(hab_h.at[ebufs[nu].at[0]], rbufs[1 - rb],
                                     gsems[1 - rb])
                pltpu.async_copy(rbufs[rb], acc.at[ebufs[u].at[1]],
                                 ssems[rb], add=True)
            return carry
        lax.fori_loop(0, T // 4, step, 0)
        drain_scatter((T - 1) % 2)
        drain_scatter(T % 2)
        plsc.subcore_barrier()

        pltpu.sync_copy(
            acc.at[pl.ds(row0, rows_per_tile), :],
            out_h.at[pl.ds(c * npad + row0, rows_per_tile), :])

    return prop_kernel


_BLK = 1000


def _mm2_body(x_ref, w2_ref, wl_ref, bl_ref, a_ref, b_ref,
              h2_ref, o0_ref):
    xb = x_ref[...]
    av = a_ref[...]
    bv = b_ref[...]
    h2 = lax.dot_general(xb, w2_ref[...], (((1,), (0,)), ((), ())),
                         preferred_element_type=jnp.float32)
    h2_ref[0] = h2 * av
    h2_ref[1] = h2 * bv
    o0_ref[...] = lax.dot_general(
        xb, wl_ref[...], (((1,), (1,)), ((), ())),
        preferred_element_type=jnp.float32) + bl_ref[...]


def _mm2(x, w2, wl, bl, a, b, npad):
    n, d = x.shape
    grid = n // _BLK
    return pl.pallas_call(
        _mm2_body,
        grid=(grid,),
        in_specs=[
            pl.BlockSpec((_BLK, d), lambda i: (i, 0)),
            pl.BlockSpec((d, d), lambda i: (0, 0)),
            pl.BlockSpec((d, d), lambda i: (0, 0)),
            pl.BlockSpec((1, d), lambda i: (0, 0)),
            pl.BlockSpec((_BLK, 1), lambda i: (i, 0)),
            pl.BlockSpec((_BLK, 1), lambda i: (i, 0)),
        ],
        out_specs=[
            pl.BlockSpec((2, _BLK, d), lambda i: (0, i, 0)),
            pl.BlockSpec((_BLK, d), lambda i: (i, 0)),
        ],
        out_shape=[
            jax.ShapeDtypeStruct((2, npad, d), jnp.float32),
            jax.ShapeDtypeStruct((n, d), jnp.float32),
        ],
    )(x, w2, wl, bl, a, b)


def _zmid_body(x_ref, w1_ref, u_ref, a_ref, b_ref, o_ref):
    av = a_ref[...]
    bv = b_ref[...]
    h1 = lax.dot_general(x_ref[...], w1_ref[...], (((1,), (0,)), ((), ())),
                         preferred_element_type=jnp.float32)
    z = h1 + bv * u_ref[0] + av * u_ref[1]
    o_ref[0] = z * av
    o_ref[1] = z * bv


def _zmid(x, w1, upair, a, b):
    n, d = x.shape
    npad = upair.shape[1]
    grid = n // _BLK
    return pl.pallas_call(
        _zmid_body,
        grid=(grid,),
        in_specs=[
            pl.BlockSpec((_BLK, d), lambda i: (i, 0)),
            pl.BlockSpec((d, d), lambda i: (0, 0)),
            pl.BlockSpec((2, _BLK, d), lambda i: (0, i, 0)),
            pl.BlockSpec((_BLK, 1), lambda i: (i, 0)),
            pl.BlockSpec((_BLK, 1), lambda i: (i, 0)),
        ],
        out_specs=pl.BlockSpec((2, _BLK, d), lambda i: (0, i, 0)),
        out_shape=jax.ShapeDtypeStruct((2, npad, d), jnp.float32),
    )(x, w1, upair, a, b)


def _fin_body(o0_ref, y_ref, a_ref, b_ref, c1_ref, c2_ref, o_ref):
    o_ref[...] = (o0_ref[...]
                  + b_ref[...] * y_ref[0] + a_ref[...] * y_ref[1]
                  + c1_ref[...] + c2_ref[...])


def _fin(o0, y, a, b, c1, c2):
    n, d = o0.shape
    grid = n // _BLK
    return pl.pallas_call(
        _fin_body,
        grid=(grid,),
        in_specs=[
            pl.BlockSpec((_BLK, d), lambda i: (i, 0)),
            pl.BlockSpec((2, _BLK, d), lambda i: (0, i, 0)),
            pl.BlockSpec((_BLK, 1), lambda i: (i, 0)),
            pl.BlockSpec((_BLK, 1), lambda i: (i, 0)),
            pl.BlockSpec((1, d), lambda i: (0, 0)),
            pl.BlockSpec((1, d), lambda i: (0, 0)),
        ],
        out_specs=pl.BlockSpec((_BLK, d), lambda i: (i, 0)),
        out_shape=jax.ShapeDtypeStruct((n, d), jnp.float32),
    )(o0, y, a, b, c1, c2)


def kernel(x, alpha, lin_w, lin_b, conv1_w, conv1_b, conv2_w, conv2_b,
           edge_index):
    n, d = x.shape
    e = edge_index.shape[1]
    e1 = e + n
    e1p = _cdiv(e1, 16 * 4 * _CH) * (16 * 4 * _CH)
    pi_iters = 20
    npad = _cdiv(n, 1024) * 1024
    t_chunks = e1p // 16 // _CH

    loops = jnp.arange(n, dtype=jnp.int32)
    row = jnp.concatenate([edge_index[0], loops])
    col = jnp.concatenate([edge_index[1], loops])
    row_p = jnp.pad(row, (0, e1p - e1)).reshape(16, t_chunks, _CH)
    col_p = jnp.pad(col, (0, e1p - e1)).reshape(16, t_chunks, _CH)
    alpha16 = jnp.full((16,), alpha, jnp.float32)

    pi_kernel = _make_pi_kernel(n, e1, e1p, pi_iters)
    a_full, b_full = pi_kernel(row_p.reshape(16, -1), col_p.reshape(16, -1),
                               alpha16)
    a2 = a_full[:n].reshape(n, 1)
    b2 = b_full[:n].reshape(n, 1)

    shp = (16, t_chunks, _CH)
    src1 = row_p
    dst1 = jnp.pad(col, (0, e1p - e1),
                   constant_values=npad - 1).reshape(shp)
    src2 = col_p + npad
    dst2 = jnp.pad(row, (0, e1p - e1),
                   constant_values=npad - 1).reshape(shp)
    ed = jnp.concatenate([
        jnp.stack([src1, dst1], axis=2),
        jnp.stack([src2, dst2], axis=2),
    ], axis=0)

    prop = _make_prop_kernel(n, d, e1p)
    c1 = conv1_b.reshape(1, d)
    c2 = conv2_b.reshape(1, d)
    bl = lin_b.reshape(1, d)

    xc = x
    for _ in range(2):
        h2ab, o0 = _mm2(xc, conv2_w, lin_w, bl, a2, b2, npad)
        u = prop(ed, h2ab.reshape(2 * npad, d)).reshape(2, npad, d)
        zab = _zmid(xc, conv1_w, u, a2, b2)
        y = prop(ed, zab.reshape(2 * npad, d)).reshape(2, npad, d)
        xc = _fin(o0, y, a2, b2, c1, c2)
    return xc
